# Initial kernel scaffold; baseline (speedup 1.0000x reference)
import jax
import jax.numpy as jnp
from jax.experimental import pallas as pl


def kernel(target_triples, pos_context, neg_context, W_target_head, W_target_tail, W_context_head, W_context_tail):
    raise NotImplementedError("write your pallas kernel here")


# speedup vs baseline: 1.6812x; 1.6812x over previous
import functools

import jax
import jax.numpy as jnp
from jax import lax
from jax.experimental import pallas as pl
from jax.experimental.pallas import tpu as pltpu
from jax.experimental.pallas import tpu_sc as plsc

EPS = 1e-15
D = 32
C = 20
L = 16
NC = 2
NS = 16
NW = NC * NS
CB = 32
IW = 80


def _sc_sums(th, tt, ph, nh, pt, nt, Wth, Wtt, Wch, Wct, B):
    nb = B // NW
    nch = nb // CB
    rpc = CB * C // IW
    rows = CB * C

    mesh = plsc.VectorSubcoreMesh(core_axis_name="c", subcore_axis_name="s")

    @functools.partial(
        pl.kernel, mesh=mesh,
        compiler_params=pltpu.CompilerParams(use_tc_tiling_on_sc=False),
        out_type=[jax.ShapeDtypeStruct((B, D), jnp.float32)] * 4,
        scratch_types=[
            pltpu.VMEM((rpc, IW), jnp.int32),
            pltpu.VMEM((rpc, IW), jnp.int32),
            pltpu.VMEM((CB,), jnp.int32),
            pltpu.VMEM((rows, D), jnp.float32),
            pltpu.VMEM((rows, D), jnp.float32),
            pltpu.VMEM((CB, D), jnp.float32),
            pltpu.VMEM((CB, D), jnp.float32),
            pltpu.VMEM((CB, D), jnp.float32),
            pltpu.SemaphoreType.DMA,
        ],
    )
    def body(th_h, tt_h, ph_h, nh_h, pt_h, nt_h, wth_h, wtt_h, wch_h, wct_h,
             o_ph, o_nh, o_pt, o_nt,
             ip_v, in_v, it_v, pr_v, nr_v, tr_v, sp_v, sn_v, sem):
        wid = lax.axis_index("s") * NC + lax.axis_index("c")
        base_b = wid * nb
        base_r = wid * (nb * C // IW)

        for (tidx_h, pidx_h, nidx_h, wt_h, wc_h, out_p, out_n) in (
            (th_h, ph_h, nh_h, wth_h, wch_h, o_ph, o_nh),
            (tt_h, pt_h, nt_h, wtt_h, wct_h, o_pt, o_nt),
        ):
            def chunk_body(g, carry):
                gb = base_b + g * CB
                r0 = base_r + g * rpc
                pltpu.sync_copy(pidx_h.at[pl.ds(r0, rpc)], ip_v)
                pltpu.sync_copy(nidx_h.at[pl.ds(r0, rpc)], in_v)
                pltpu.sync_copy(tidx_h.at[pl.ds(gb, CB)], it_v)
                cps = []
                for j in range(rpc):
                    cps.append(pltpu.async_copy(
                        wc_h.at[ip_v.at[j]], pr_v.at[pl.ds(j * IW, IW)], sem))
                for j in range(rpc):
                    cps.append(pltpu.async_copy(
                        wc_h.at[in_v.at[j]], nr_v.at[pl.ds(j * IW, IW)], sem))
                cps.append(pltpu.async_copy(wt_h.at[it_v], tr_v, sem))
                for cp in cps:
                    cp.wait()

                def b_body(b, carry2):
                    r = b * C
                    p0 = pr_v[r, pl.ds(0, L)]
                    p1 = pr_v[r, pl.ds(L, L)]
                    n0 = nr_v[r, pl.ds(0, L)]
                    n1 = nr_v[r, pl.ds(L, L)]
                    for c in range(1, C):
                        p0 = p0 + pr_v[r + c, pl.ds(0, L)]
                        p1 = p1 + pr_v[r + c, pl.ds(L, L)]
                        n0 = n0 + nr_v[r + c, pl.ds(0, L)]
                        n1 = n1 + nr_v[r + c, pl.ds(L, L)]
                    t0 = tr_v[b, pl.ds(0, L)]
                    t1 = tr_v[b, pl.ds(L, L)]
                    sp_v[b, pl.ds(0, L)] = t0 * p0
                    sp_v[b, pl.ds(L, L)] = t1 * p1
                    sn_v[b, pl.ds(0, L)] = t0 * n0
                    sn_v[b, pl.ds(L, L)] = t1 * n1
                    return carry2

                lax.fori_loop(0, CB, b_body, 0)
                pltpu.sync_copy(sp_v, out_p.at[pl.ds(gb, CB)])
                pltpu.sync_copy(sn_v, out_n.at[pl.ds(gb, CB)])
                return carry

            lax.fori_loop(0, nch, chunk_body, 0)

    return body(th, tt, ph, nh, pt, nt, Wth, Wtt, Wch, Wct)


def _softplus(z):
    return jnp.maximum(z, 0.0) + jnp.log1p(jnp.exp(-jnp.abs(z)))


def _tc_loss(ph, nh, pt, nt, n_elems):
    def body(ph_r, nh_r, pt_r, nt_r, o_r):
        sh = jnp.sum(_softplus(-(ph_r[...] + EPS))
                     + _softplus(-(1.0 - (nh_r[...] + EPS))))
        st = jnp.sum(_softplus(-(pt_r[...] + EPS))
                     + _softplus(-(1.0 - (nt_r[...] + EPS))))
        o_r[...] = ((sh + st) * (1.0 / n_elems))[None, None]

    out = pl.pallas_call(
        body, out_shape=jax.ShapeDtypeStruct((1, 1), jnp.float32),
    )(ph, nh, pt, nt)
    return out[0, 0]


def kernel(target_triples, pos_context, neg_context,
           W_target_head, W_target_tail, W_context_head, W_context_tail):
    B = target_triples.shape[0]
    th = target_triples[:, 0].astype(jnp.int32)
    tt = target_triples[:, 2].astype(jnp.int32)
    ph = pos_context[:, :, 0].astype(jnp.int32).reshape(-1, IW)
    pt = pos_context[:, :, 2].astype(jnp.int32).reshape(-1, IW)
    nh = neg_context[:, :, 0].astype(jnp.int32).reshape(-1, IW)
    nt = neg_context[:, :, 2].astype(jnp.int32).reshape(-1, IW)

    o_ph, o_nh, o_pt, o_nt = _sc_sums(
        th, tt, ph, nh, pt, nt,
        W_target_head, W_target_tail, W_context_head, W_context_tail, B)

    r = B * D // 128
    return _tc_loss(o_ph.reshape(r, 128), o_nh.reshape(r, 128),
                    o_pt.reshape(r, 128), o_nt.reshape(r, 128), B * D)
